# Initial kernel scaffold; baseline (speedup 1.0000x reference)
import jax
import jax.numpy as jnp
from jax.experimental import pallas as pl


def kernel(x, edge_index, W1, b1, W2, b2, Wc, bc):
    raise NotImplementedError("write your pallas kernel here")


# speedup vs baseline: 3.0662x; 3.0662x over previous
import dataclasses
import functools

import jax
import jax.numpy as jnp
from jax import lax
from jax.experimental import pallas as pl
from jax.experimental.pallas import tpu as pltpu
from jax.experimental.pallas import tpu_sc as plsc

N = 10000
E = 320000
D_IN = 128
D_HID = 128
D_OUT = 64

NC = 2
NS = 16
NW = NC * NS

NPAD = 10240
EPAD = 327680
EPW = EPAD // NW
CH = 128
RPS = NPAD // NS

_mesh = plsc.VectorSubcoreMesh(core_axis_name="c", subcore_axis_name="s")

_sc_params = pltpu.CompilerParams()
if "needs_layout_passes" in pltpu.CompilerParams.__dataclass_fields__:
    _sc_params = dataclasses.replace(_sc_params, needs_layout_passes=False)


def _deg_body(srcp_hbm, dstp_hbm, out_hbm, sidx_v, didx_v, hist_s, hist_d):
    cid = lax.axis_index("c")
    sid = lax.axis_index("s")
    wid = cid * NS + sid
    z16 = jnp.zeros((16,), jnp.float32)
    o16 = jnp.ones((16,), jnp.float32)

    @pl.loop(0, NPAD, step=16)
    def _zero(k):
        hist_s[pl.ds(k, 16)] = z16
        hist_d[pl.ds(k, 16)] = z16

    base = wid * EPW

    @pl.loop(0, EPW, step=CH)
    def _chunk(i):
        pltpu.sync_copy(srcp_hbm.at[pl.ds(base + i, CH)], sidx_v)
        pltpu.sync_copy(dstp_hbm.at[pl.ds(base + i, CH)], didx_v)

        @pl.loop(0, CH, step=16)
        def _vec(j):
            plsc.addupdate_scatter(hist_s, [sidx_v[pl.ds(j, 16)]], o16)
            plsc.addupdate_scatter(hist_d, [didx_v[pl.ds(j, 16)]], o16)

    pltpu.sync_copy(hist_s, out_hbm.at[wid, 0])
    pltpu.sync_copy(hist_d, out_hbm.at[wid, 1])


_deg_call = pl.kernel(
    _deg_body,
    out_type=jax.ShapeDtypeStruct((NW, 2, NPAD), jnp.float32),
    mesh=_mesh,
    scratch_types=[
        pltpu.VMEM((CH,), jnp.int32),
        pltpu.VMEM((CH,), jnp.int32),
        pltpu.VMEM((NPAD,), jnp.float32),
        pltpu.VMEM((NPAD,), jnp.float32),
    ],
    compiler_params=_sc_params,
)


def _edge_body(h_hbm, srcp_hbm, dstp_hbm, out_hbm,
               acc_sh, sidx_v, didx_v, rows_v, sem):
    cid = lax.axis_index("c")
    sid = lax.axis_index("s")
    wid = cid * NS + sid
    z16 = jnp.zeros((16,), jnp.float32)

    @pl.loop(0, CH)
    def _zr(r):
        @pl.loop(0, D_HID, step=16)
        def _zc(c):
            rows_v[r, pl.ds(c, 16)] = z16

    @pl.loop(0, RPS, step=CH)
    def _za(r):
        pltpu.sync_copy(rows_v, acc_sh.at[pl.ds(sid * RPS + r, CH)])

    plsc.subcore_barrier()

    base = wid * EPW

    @pl.loop(0, EPW, step=CH)
    def _chunk(i):
        pltpu.sync_copy(srcp_hbm.at[pl.ds(base + i, CH)], sidx_v)
        pltpu.sync_copy(dstp_hbm.at[pl.ds(base + i, CH)], didx_v)
        pltpu.async_copy(h_hbm.at[sidx_v], rows_v, sem).wait()
        pltpu.sync_copy(rows_v, acc_sh.at[didx_v], add=True)

    plsc.subcore_barrier()

    @pl.loop(0, RPS, step=CH)
    def _drain(r):
        rr = sid * RPS + r
        pltpu.sync_copy(acc_sh.at[pl.ds(rr, CH)], out_hbm.at[cid, pl.ds(rr, CH)])


_edge_call = pl.kernel(
    _edge_body,
    out_type=jax.ShapeDtypeStruct((NC, NPAD, D_HID), jnp.float32),
    mesh=_mesh,
    scratch_types=[
        pltpu.VMEM_SHARED((NPAD, D_HID), jnp.float32),
        pltpu.VMEM((CH,), jnp.int32),
        pltpu.VMEM((CH,), jnp.int32),
        pltpu.VMEM((CH, D_HID), jnp.float32),
        pltpu.SemaphoreType.DMA,
    ],
    compiler_params=_sc_params,
)


TBLK = 1024
_PREC = lax.Precision.HIGHEST


def _mm_body(x_ref, w_ref, o_ref):
    o_ref[...] = jnp.dot(x_ref[...], w_ref[...],
                         preferred_element_type=jnp.float32, precision=_PREC)


_mm_call = pl.pallas_call(
    _mm_body,
    grid=(NPAD // TBLK,),
    in_specs=[
        pl.BlockSpec((TBLK, D_IN), lambda i: (i, 0)),
        pl.BlockSpec((D_IN, D_HID), lambda i: (0, 0)),
    ],
    out_specs=pl.BlockSpec((TBLK, D_HID), lambda i: (i, 0)),
    out_shape=jax.ShapeDtypeStruct((NPAD, D_HID), jnp.float32),
)


def _norm_scale_body(xw_ref, hist_ref, h1s_ref, ns_ref, nd_ref):
    deg = jnp.sum(hist_ref[...], axis=0)
    norm = jnp.where(deg > 0.0, lax.rsqrt(jnp.maximum(deg, 1.0)), 0.0)
    ns2d = jnp.broadcast_to(norm[0][:, None], (TBLK, D_HID))
    nd2d = jnp.broadcast_to(norm[1][:, None], (TBLK, D_HID))
    ns_ref[...] = ns2d
    nd_ref[...] = nd2d
    h1s_ref[...] = xw_ref[...] * ns2d


_norm_scale_call = pl.pallas_call(
    _norm_scale_body,
    grid=(NPAD // TBLK,),
    in_specs=[
        pl.BlockSpec((TBLK, D_HID), lambda i: (i, 0)),
        pl.BlockSpec((NW, 2, TBLK), lambda i: (0, 0, i)),
    ],
    out_specs=[
        pl.BlockSpec((TBLK, D_HID), lambda i: (i, 0)),
        pl.BlockSpec((TBLK, D_HID), lambda i: (i, 0)),
        pl.BlockSpec((TBLK, D_HID), lambda i: (i, 0)),
    ],
    out_shape=[
        jax.ShapeDtypeStruct((NPAD, D_HID), jnp.float32),
        jax.ShapeDtypeStruct((NPAD, D_HID), jnp.float32),
        jax.ShapeDtypeStruct((NPAD, D_HID), jnp.float32),
    ],
)


def _mid_body(p_ref, nd_ref, ns_ref, b_ref, w_ref, o_ref):
    agg = p_ref[0] + p_ref[1]
    h = jnp.maximum(agg * nd_ref[...] + b_ref[...], 0.0)
    o_ref[...] = jnp.dot(h, w_ref[...], preferred_element_type=jnp.float32,
                         precision=_PREC) * ns_ref[...]


_mid_call = pl.pallas_call(
    _mid_body,
    grid=(NPAD // TBLK,),
    in_specs=[
        pl.BlockSpec((NC, TBLK, D_HID), lambda i: (0, i, 0)),
        pl.BlockSpec((TBLK, D_HID), lambda i: (i, 0)),
        pl.BlockSpec((TBLK, D_HID), lambda i: (i, 0)),
        pl.BlockSpec((D_HID,), lambda i: (0,)),
        pl.BlockSpec((D_HID, D_HID), lambda i: (0, 0)),
    ],
    out_specs=pl.BlockSpec((TBLK, D_HID), lambda i: (i, 0)),
    out_shape=jax.ShapeDtypeStruct((NPAD, D_HID), jnp.float32),
)


def _final_body(p_ref, nd_ref, b_ref, wc_ref, bc_ref, o_ref):
    agg = p_ref[0] + p_ref[1]
    h = jnp.maximum(agg * nd_ref[...] + b_ref[...], 0.0)
    o_ref[...] = jnp.dot(h, wc_ref[...], preferred_element_type=jnp.float32,
                         precision=_PREC) + bc_ref[...]


_final_call = pl.pallas_call(
    _final_body,
    grid=(NPAD // TBLK,),
    in_specs=[
        pl.BlockSpec((NC, TBLK, D_HID), lambda i: (0, i, 0)),
        pl.BlockSpec((TBLK, D_HID), lambda i: (i, 0)),
        pl.BlockSpec((D_HID,), lambda i: (0,)),
        pl.BlockSpec((D_HID, D_HID), lambda i: (0, 0)),
        pl.BlockSpec((D_HID,), lambda i: (0,)),
    ],
    out_specs=pl.BlockSpec((TBLK, D_HID), lambda i: (i, 0)),
    out_shape=jax.ShapeDtypeStruct((NPAD, D_HID), jnp.float32),
)


def kernel(x, edge_index, W1, b1, W2, b2, Wc, bc):
    x_pad = jnp.zeros((NPAD, D_IN), jnp.float32).at[:N].set(x)
    fill = jnp.full((EPAD - E,), NPAD - 1, jnp.int32)
    srcp = jnp.concatenate([edge_index[0], fill])
    dstp = jnp.concatenate([edge_index[1], fill])
    wc_pad = jnp.zeros((D_HID, D_HID), jnp.float32).at[:, :D_OUT].set(Wc)
    bc_pad = jnp.zeros((D_HID,), jnp.float32).at[:D_OUT].set(bc)

    hists = _deg_call(srcp, dstp)
    xw = _mm_call(x_pad, W1)
    h1s, ns2d, nd2d = _norm_scale_call(xw, hists)

    p1 = _edge_call(h1s, srcp, dstp)
    h2s = _mid_call(p1, nd2d, ns2d, b1, W2)
    p2 = _edge_call(h2s, srcp, dstp)
    outp = _final_call(p2, nd2d, b2, wc_pad, bc_pad)
    return outp[:N, :D_OUT]


# speedup vs baseline: 8.7912x; 2.8672x over previous
import dataclasses
import functools

import jax
import jax.numpy as jnp
from jax import lax
from jax.experimental import pallas as pl
from jax.experimental.pallas import tpu as pltpu
from jax.experimental.pallas import tpu_sc as plsc

N = 10000
E = 320000
D_IN = 128
D_HID = 128
D_OUT = 64

NC = 2
NS = 16
NW = NC * NS

NPAD = 10240
EPAD = 327680
EPW = EPAD // NW
CH = 128
RPS = NPAD // NS

_mesh = plsc.VectorSubcoreMesh(core_axis_name="c", subcore_axis_name="s")

_sc_params = pltpu.CompilerParams()
if "needs_layout_passes" in pltpu.CompilerParams.__dataclass_fields__:
    _sc_params = dataclasses.replace(_sc_params, needs_layout_passes=False)


def _deg_body(srcp_hbm, dstp_hbm, out_hbm, sidx_v, didx_v, hist_s, hist_d):
    cid = lax.axis_index("c")
    sid = lax.axis_index("s")
    wid = cid * NS + sid
    z16 = jnp.zeros((16,), jnp.float32)
    o16 = jnp.ones((16,), jnp.float32)

    @pl.loop(0, NPAD, step=16)
    def _zero(k):
        hist_s[pl.ds(k, 16)] = z16
        hist_d[pl.ds(k, 16)] = z16

    base = wid * EPW

    @pl.loop(0, EPW, step=CH)
    def _chunk(i):
        pltpu.sync_copy(srcp_hbm.at[pl.ds(base + i, CH)], sidx_v)
        pltpu.sync_copy(dstp_hbm.at[pl.ds(base + i, CH)], didx_v)

        @pl.loop(0, CH, step=16)
        def _vec(j):
            plsc.addupdate_scatter(hist_s, [sidx_v[pl.ds(j, 16)]], o16)
            plsc.addupdate_scatter(hist_d, [didx_v[pl.ds(j, 16)]], o16)

    pltpu.sync_copy(hist_s, out_hbm.at[wid, 0])
    pltpu.sync_copy(hist_d, out_hbm.at[wid, 1])


_deg_call = pl.kernel(
    _deg_body,
    out_type=jax.ShapeDtypeStruct((NW, 2, NPAD), jnp.float32),
    mesh=_mesh,
    scratch_types=[
        pltpu.VMEM((CH,), jnp.int32),
        pltpu.VMEM((CH,), jnp.int32),
        pltpu.VMEM((NPAD,), jnp.float32),
        pltpu.VMEM((NPAD,), jnp.float32),
    ],
    compiler_params=_sc_params,
)


NCHUNK = EPW // CH
NHALF = 2
NCH = NCHUNK // NHALF


def _edge_body(h_hbm, srcp_hbm, dstp_hbm, out_hbm,
               acc_sh, sidx_v, didx_v, rows0, rows1,
               gsem0, gsem1, ssem0, ssem1):
    cid = lax.axis_index("c")
    sid = lax.axis_index("s")
    wid = cid * NS + sid
    z16 = jnp.zeros((16,), jnp.float32)

    @pl.loop(0, CH)
    def _zr(r):
        @pl.loop(0, D_HID, step=16)
        def _zc(c):
            rows0[r, pl.ds(c, 16)] = z16

    @pl.loop(0, RPS, step=CH)
    def _za(r):
        pltpu.sync_copy(rows0, acc_sh.at[pl.ds(sid * RPS + r, CH)])

    plsc.subcore_barrier()

    for hf in range(NHALF):
        pltpu.sync_copy(srcp_hbm.at[wid, hf], sidx_v)
        pltpu.sync_copy(dstp_hbm.at[wid, hf], didx_v)
        pltpu.async_copy(h_hbm.at[sidx_v.at[0]], rows0, gsem0)
        pltpu.async_copy(h_hbm.at[sidx_v.at[1]], rows1, gsem1)

        @pl.loop(0, NCH, step=2)
        def _chunk(g):
            pltpu.make_async_copy(h_hbm.at[sidx_v.at[g]], rows0, gsem0).wait()
            pltpu.async_copy(rows0, acc_sh.at[didx_v.at[g]], ssem0, add=True)
            pltpu.make_async_copy(h_hbm.at[sidx_v.at[g + 1]], rows1, gsem1).wait()
            pltpu.async_copy(rows1, acc_sh.at[didx_v.at[g + 1]], ssem1, add=True)

            @pl.when(g + 2 < NCH)
            def _n0():
                pltpu.make_async_copy(rows0, acc_sh.at[didx_v.at[g]], ssem0).wait()
                pltpu.async_copy(h_hbm.at[sidx_v.at[g + 2]], rows0, gsem0)

            @pl.when(g + 3 < NCH)
            def _n1():
                pltpu.make_async_copy(rows1, acc_sh.at[didx_v.at[g + 1]], ssem1).wait()
                pltpu.async_copy(h_hbm.at[sidx_v.at[g + 3]], rows1, gsem1)

        pltpu.make_async_copy(rows0, acc_sh.at[didx_v.at[NCH - 2]], ssem0).wait()
        pltpu.make_async_copy(rows1, acc_sh.at[didx_v.at[NCH - 1]], ssem1).wait()

    plsc.subcore_barrier()

    @pl.loop(0, RPS, step=CH)
    def _drain(r):
        rr = sid * RPS + r
        pltpu.sync_copy(acc_sh.at[pl.ds(rr, CH)], out_hbm.at[cid, pl.ds(rr, CH)])


_edge_call = pl.kernel(
    _edge_body,
    out_type=jax.ShapeDtypeStruct((NC, NPAD, D_HID), jnp.float32),
    mesh=_mesh,
    scratch_types=[
        pltpu.VMEM_SHARED((NPAD, D_HID), jnp.float32),
        pltpu.VMEM((NCH, CH), jnp.int32),
        pltpu.VMEM((NCH, CH), jnp.int32),
        pltpu.VMEM((CH, D_HID), jnp.float32),
        pltpu.VMEM((CH, D_HID), jnp.float32),
        pltpu.SemaphoreType.DMA,
        pltpu.SemaphoreType.DMA,
        pltpu.SemaphoreType.DMA,
        pltpu.SemaphoreType.DMA,
    ],
    compiler_params=_sc_params,
)


TBLK = 1024
_PREC = lax.Precision.HIGHEST


def _mm_body(x_ref, w_ref, o_ref):
    o_ref[...] = jnp.dot(x_ref[...], w_ref[...],
                         preferred_element_type=jnp.float32, precision=_PREC)


_mm_call = pl.pallas_call(
    _mm_body,
    grid=(NPAD // TBLK,),
    in_specs=[
        pl.BlockSpec((TBLK, D_IN), lambda i: (i, 0)),
        pl.BlockSpec((D_IN, D_HID), lambda i: (0, 0)),
    ],
    out_specs=pl.BlockSpec((TBLK, D_HID), lambda i: (i, 0)),
    out_shape=jax.ShapeDtypeStruct((NPAD, D_HID), jnp.float32),
)


def _norm_scale_body(xw_ref, hist_ref, h1s_ref, ns_ref, nd_ref):
    deg = jnp.sum(hist_ref[...], axis=0)
    norm = jnp.where(deg > 0.0, lax.rsqrt(jnp.maximum(deg, 1.0)), 0.0)
    ns2d = jnp.broadcast_to(norm[0][:, None], (TBLK, D_HID))
    nd2d = jnp.broadcast_to(norm[1][:, None], (TBLK, D_HID))
    ns_ref[...] = ns2d
    nd_ref[...] = nd2d
    h1s_ref[...] = xw_ref[...] * ns2d


_norm_scale_call = pl.pallas_call(
    _norm_scale_body,
    grid=(NPAD // TBLK,),
    in_specs=[
        pl.BlockSpec((TBLK, D_HID), lambda i: (i, 0)),
        pl.BlockSpec((NW, 2, TBLK), lambda i: (0, 0, i)),
    ],
    out_specs=[
        pl.BlockSpec((TBLK, D_HID), lambda i: (i, 0)),
        pl.BlockSpec((TBLK, D_HID), lambda i: (i, 0)),
        pl.BlockSpec((TBLK, D_HID), lambda i: (i, 0)),
    ],
    out_shape=[
        jax.ShapeDtypeStruct((NPAD, D_HID), jnp.float32),
        jax.ShapeDtypeStruct((NPAD, D_HID), jnp.float32),
        jax.ShapeDtypeStruct((NPAD, D_HID), jnp.float32),
    ],
)


def _mid_body(p_ref, nd_ref, ns_ref, b_ref, w_ref, o_ref):
    agg = p_ref[0] + p_ref[1]
    h = jnp.maximum(agg * nd_ref[...] + b_ref[...], 0.0)
    o_ref[...] = jnp.dot(h, w_ref[...], preferred_element_type=jnp.float32,
                         precision=_PREC) * ns_ref[...]


_mid_call = pl.pallas_call(
    _mid_body,
    grid=(NPAD // TBLK,),
    in_specs=[
        pl.BlockSpec((NC, TBLK, D_HID), lambda i: (0, i, 0)),
        pl.BlockSpec((TBLK, D_HID), lambda i: (i, 0)),
        pl.BlockSpec((TBLK, D_HID), lambda i: (i, 0)),
        pl.BlockSpec((D_HID,), lambda i: (0,)),
        pl.BlockSpec((D_HID, D_HID), lambda i: (0, 0)),
    ],
    out_specs=pl.BlockSpec((TBLK, D_HID), lambda i: (i, 0)),
    out_shape=jax.ShapeDtypeStruct((NPAD, D_HID), jnp.float32),
)


def _final_body(p_ref, nd_ref, b_ref, wc_ref, bc_ref, o_ref):
    agg = p_ref[0] + p_ref[1]
    h = jnp.maximum(agg * nd_ref[...] + b_ref[...], 0.0)
    o_ref[...] = jnp.dot(h, wc_ref[...], preferred_element_type=jnp.float32,
                         precision=_PREC) + bc_ref[...]


_final_call = pl.pallas_call(
    _final_body,
    grid=(NPAD // TBLK,),
    in_specs=[
        pl.BlockSpec((NC, TBLK, D_HID), lambda i: (0, i, 0)),
        pl.BlockSpec((TBLK, D_HID), lambda i: (i, 0)),
        pl.BlockSpec((D_HID,), lambda i: (0,)),
        pl.BlockSpec((D_HID, D_HID), lambda i: (0, 0)),
        pl.BlockSpec((D_HID,), lambda i: (0,)),
    ],
    out_specs=pl.BlockSpec((TBLK, D_HID), lambda i: (i, 0)),
    out_shape=jax.ShapeDtypeStruct((NPAD, D_HID), jnp.float32),
)


def kernel(x, edge_index, W1, b1, W2, b2, Wc, bc):
    x_pad = jnp.zeros((NPAD, D_IN), jnp.float32).at[:N].set(x)
    fill = N + (jnp.arange(EPAD - E, dtype=jnp.int32) % (NPAD - N))
    srcp = jnp.concatenate([edge_index[0], fill])
    dstp = jnp.concatenate([edge_index[1], fill])
    srcp3 = srcp.reshape(NW, NHALF, NCH, CH)
    dstp3 = dstp.reshape(NW, NHALF, NCH, CH)
    wc_pad = jnp.zeros((D_HID, D_HID), jnp.float32).at[:, :D_OUT].set(Wc)
    bc_pad = jnp.zeros((D_HID,), jnp.float32).at[:D_OUT].set(bc)

    hists = _deg_call(srcp, dstp)
    xw = _mm_call(x_pad, W1)
    h1s, ns2d, nd2d = _norm_scale_call(xw, hists)

    p1 = _edge_call(h1s, srcp3, dstp3)
    h2s = _mid_call(p1, nd2d, ns2d, b1, W2)
    p2 = _edge_call(h2s, srcp3, dstp3)
    outp = _final_call(p2, nd2d, b2, wc_pad, bc_pad)
    return outp[:N, :D_OUT]


# speedup vs baseline: 10.4061x; 1.1837x over previous
import dataclasses
import functools

import jax
import jax.numpy as jnp
from jax import lax
from jax.experimental import pallas as pl
from jax.experimental.pallas import tpu as pltpu
from jax.experimental.pallas import tpu_sc as plsc

N = 10000
E = 320000
D_IN = 128
D_HID = 128
D_OUT = 64

NC = 2
NS = 16
NW = NC * NS

NPAD = 10240
EPAD = 327680
EPW = EPAD // NW
CH = 128
RPS = NPAD // NS

_mesh = plsc.VectorSubcoreMesh(core_axis_name="c", subcore_axis_name="s")

_sc_params = pltpu.CompilerParams()
if "needs_layout_passes" in pltpu.CompilerParams.__dataclass_fields__:
    _sc_params = dataclasses.replace(_sc_params, needs_layout_passes=False)


def _deg_body(srcp_hbm, dstp_hbm, out_hbm, sidx_v, didx_v, hist_s, hist_d):
    cid = lax.axis_index("c")
    sid = lax.axis_index("s")
    wid = cid * NS + sid
    z16 = jnp.zeros((16,), jnp.float32)
    o16 = jnp.ones((16,), jnp.float32)

    pltpu.sync_copy(srcp_hbm.at[wid], sidx_v)
    pltpu.sync_copy(dstp_hbm.at[wid], didx_v)

    @pl.loop(0, NPAD, step=16)
    def _zero(k):
        hist_s[pl.ds(k, 16)] = z16
        hist_d[pl.ds(k, 16)] = z16

    @pl.loop(0, EPW, step=16)
    def _vec(j):
        plsc.addupdate_scatter(hist_s, [sidx_v[pl.ds(j, 16)]], o16)
        plsc.addupdate_scatter(hist_d, [didx_v[pl.ds(j, 16)]], o16)

    pltpu.sync_copy(hist_s, out_hbm.at[wid, 0])
    pltpu.sync_copy(hist_d, out_hbm.at[wid, 1])


_deg_call = pl.kernel(
    _deg_body,
    out_type=jax.ShapeDtypeStruct((NW, 2, NPAD), jnp.float32),
    mesh=_mesh,
    scratch_types=[
        pltpu.VMEM((EPW,), jnp.int32),
        pltpu.VMEM((EPW,), jnp.int32),
        pltpu.VMEM((NPAD,), jnp.float32),
        pltpu.VMEM((NPAD,), jnp.float32),
    ],
    compiler_params=_sc_params,
)


NCHUNK = EPW // CH
NHALF = 2
NCH = NCHUNK // NHALF


def _edge_body(h_hbm, srcp_hbm, dstp_hbm, out_hbm,
               acc_sh, sidx_v, didx_v, rows0, rows1,
               gsem0, gsem1, ssem0, ssem1):
    cid = lax.axis_index("c")
    sid = lax.axis_index("s")
    wid = cid * NS + sid
    z16 = jnp.zeros((16,), jnp.float32)

    @pl.loop(0, CH)
    def _zr(r):
        @pl.loop(0, D_HID, step=16)
        def _zc(c):
            rows0[r, pl.ds(c, 16)] = z16

    @pl.loop(0, RPS, step=CH)
    def _za(r):
        pltpu.sync_copy(rows0, acc_sh.at[pl.ds(sid * RPS + r, CH)])

    plsc.subcore_barrier()

    for hf in range(NHALF):
        pltpu.sync_copy(srcp_hbm.at[wid, hf], sidx_v)
        pltpu.sync_copy(dstp_hbm.at[wid, hf], didx_v)
        pltpu.async_copy(h_hbm.at[sidx_v.at[0]], rows0, gsem0)
        pltpu.async_copy(h_hbm.at[sidx_v.at[1]], rows1, gsem1)

        @pl.loop(0, NCH, step=2)
        def _chunk(g):
            pltpu.make_async_copy(h_hbm.at[sidx_v.at[g]], rows0, gsem0).wait()
            pltpu.async_copy(rows0, acc_sh.at[didx_v.at[g]], ssem0, add=True)
            pltpu.make_async_copy(h_hbm.at[sidx_v.at[g + 1]], rows1, gsem1).wait()
            pltpu.async_copy(rows1, acc_sh.at[didx_v.at[g + 1]], ssem1, add=True)

            @pl.when(g + 2 < NCH)
            def _n0():
                pltpu.make_async_copy(rows0, acc_sh.at[didx_v.at[g]], ssem0).wait()
                pltpu.async_copy(h_hbm.at[sidx_v.at[g + 2]], rows0, gsem0)

            @pl.when(g + 3 < NCH)
            def _n1():
                pltpu.make_async_copy(rows1, acc_sh.at[didx_v.at[g + 1]], ssem1).wait()
                pltpu.async_copy(h_hbm.at[sidx_v.at[g + 3]], rows1, gsem1)

        pltpu.make_async_copy(rows0, acc_sh.at[didx_v.at[NCH - 2]], ssem0).wait()
        pltpu.make_async_copy(rows1, acc_sh.at[didx_v.at[NCH - 1]], ssem1).wait()

    plsc.subcore_barrier()

    @pl.loop(0, RPS, step=CH)
    def _drain(r):
        rr = sid * RPS + r
        pltpu.sync_copy(acc_sh.at[pl.ds(rr, CH)], out_hbm.at[cid, pl.ds(rr, CH)])


_edge_call = pl.kernel(
    _edge_body,
    out_type=jax.ShapeDtypeStruct((NC, NPAD, D_HID), jnp.float32),
    mesh=_mesh,
    scratch_types=[
        pltpu.VMEM_SHARED((NPAD, D_HID), jnp.float32),
        pltpu.VMEM((NCH, CH), jnp.int32),
        pltpu.VMEM((NCH, CH), jnp.int32),
        pltpu.VMEM((CH, D_HID), jnp.float32),
        pltpu.VMEM((CH, D_HID), jnp.float32),
        pltpu.SemaphoreType.DMA,
        pltpu.SemaphoreType.DMA,
        pltpu.SemaphoreType.DMA,
        pltpu.SemaphoreType.DMA,
    ],
    compiler_params=_sc_params,
)


TBLK = 1024
_PREC = lax.Precision.HIGHEST


def _mm_body(x_ref, w_ref, o_ref):
    o_ref[...] = jnp.dot(x_ref[...], w_ref[...],
                         preferred_element_type=jnp.float32, precision=_PREC)


_mm_call = pl.pallas_call(
    _mm_body,
    grid=(NPAD // TBLK,),
    in_specs=[
        pl.BlockSpec((TBLK, D_IN), lambda i: (i, 0)),
        pl.BlockSpec((D_IN, D_HID), lambda i: (0, 0)),
    ],
    out_specs=pl.BlockSpec((TBLK, D_HID), lambda i: (i, 0)),
    out_shape=jax.ShapeDtypeStruct((NPAD, D_HID), jnp.float32),
)


def _norm_scale_body(xw_ref, hist_ref, h1s_ref, ns_ref, nd_ref):
    deg = jnp.sum(hist_ref[...], axis=0)
    norm = jnp.where(deg > 0.0, lax.rsqrt(jnp.maximum(deg, 1.0)), 0.0)
    ns2d = jnp.broadcast_to(norm[0][:, None], (TBLK, D_HID))
    nd2d = jnp.broadcast_to(norm[1][:, None], (TBLK, D_HID))
    ns_ref[...] = ns2d
    nd_ref[...] = nd2d
    h1s_ref[...] = xw_ref[...] * ns2d


_norm_scale_call = pl.pallas_call(
    _norm_scale_body,
    grid=(NPAD // TBLK,),
    in_specs=[
        pl.BlockSpec((TBLK, D_HID), lambda i: (i, 0)),
        pl.BlockSpec((NW, 2, TBLK), lambda i: (0, 0, i)),
    ],
    out_specs=[
        pl.BlockSpec((TBLK, D_HID), lambda i: (i, 0)),
        pl.BlockSpec((TBLK, D_HID), lambda i: (i, 0)),
        pl.BlockSpec((TBLK, D_HID), lambda i: (i, 0)),
    ],
    out_shape=[
        jax.ShapeDtypeStruct((NPAD, D_HID), jnp.float32),
        jax.ShapeDtypeStruct((NPAD, D_HID), jnp.float32),
        jax.ShapeDtypeStruct((NPAD, D_HID), jnp.float32),
    ],
)


def _mid_body(p_ref, nd_ref, ns_ref, b_ref, w_ref, o_ref):
    agg = p_ref[0] + p_ref[1]
    h = jnp.maximum(agg * nd_ref[...] + b_ref[...], 0.0)
    o_ref[...] = jnp.dot(h, w_ref[...], preferred_element_type=jnp.float32,
                         precision=_PREC) * ns_ref[...]


_mid_call = pl.pallas_call(
    _mid_body,
    grid=(NPAD // TBLK,),
    in_specs=[
        pl.BlockSpec((NC, TBLK, D_HID), lambda i: (0, i, 0)),
        pl.BlockSpec((TBLK, D_HID), lambda i: (i, 0)),
        pl.BlockSpec((TBLK, D_HID), lambda i: (i, 0)),
        pl.BlockSpec((D_HID,), lambda i: (0,)),
        pl.BlockSpec((D_HID, D_HID), lambda i: (0, 0)),
    ],
    out_specs=pl.BlockSpec((TBLK, D_HID), lambda i: (i, 0)),
    out_shape=jax.ShapeDtypeStruct((NPAD, D_HID), jnp.float32),
)


def _final_body(p_ref, nd_ref, b_ref, wc_ref, bc_ref, o_ref):
    agg = p_ref[0] + p_ref[1]
    h = jnp.maximum(agg * nd_ref[...] + b_ref[...], 0.0)
    o_ref[...] = jnp.dot(h, wc_ref[...], preferred_element_type=jnp.float32,
                         precision=_PREC) + bc_ref[...]


_final_call = pl.pallas_call(
    _final_body,
    grid=(NPAD // TBLK,),
    in_specs=[
        pl.BlockSpec((NC, TBLK, D_HID), lambda i: (0, i, 0)),
        pl.BlockSpec((TBLK, D_HID), lambda i: (i, 0)),
        pl.BlockSpec((D_HID,), lambda i: (0,)),
        pl.BlockSpec((D_HID, D_HID), lambda i: (0, 0)),
        pl.BlockSpec((D_HID,), lambda i: (0,)),
    ],
    out_specs=pl.BlockSpec((TBLK, D_HID), lambda i: (i, 0)),
    out_shape=jax.ShapeDtypeStruct((NPAD, D_HID), jnp.float32),
)


def kernel(x, edge_index, W1, b1, W2, b2, Wc, bc):
    x_pad = jnp.zeros((NPAD, D_IN), jnp.float32).at[:N].set(x)
    fill = N + (jnp.arange(EPAD - E, dtype=jnp.int32) % (NPAD - N))
    srcp = jnp.concatenate([edge_index[0], fill])
    dstp = jnp.concatenate([edge_index[1], fill])
    srcp2 = srcp.reshape(NW, EPW)
    dstp2 = dstp.reshape(NW, EPW)
    srcp3 = srcp.reshape(NW, NHALF, NCH, CH)
    dstp3 = dstp.reshape(NW, NHALF, NCH, CH)
    wc_pad = jnp.zeros((D_HID, D_HID), jnp.float32).at[:, :D_OUT].set(Wc)
    bc_pad = jnp.zeros((D_HID,), jnp.float32).at[:D_OUT].set(bc)

    hists = _deg_call(srcp2, dstp2)
    xw = _mm_call(x_pad, W1)
    h1s, ns2d, nd2d = _norm_scale_call(xw, hists)

    p1 = _edge_call(h1s, srcp3, dstp3)
    h2s = _mid_call(p1, nd2d, ns2d, b1, W2)
    p2 = _edge_call(h2s, srcp3, dstp3)
    outp = _final_call(p2, nd2d, b2, wc_pad, bc_pad)
    return outp[:N, :D_OUT]
